# Initial kernel scaffold; baseline (speedup 1.0000x reference)
import jax
import jax.numpy as jnp
from jax.experimental import pallas as pl


def kernel(x, W, gamma, beta, edge_idx, edge_type):
    raise NotImplementedError("write your pallas kernel here")


# speedup vs baseline: 4.7162x; 4.7162x over previous
import functools

import jax
import jax.numpy as jnp
from jax import lax
from jax.experimental import pallas as pl
from jax.experimental.pallas import tpu as pltpu
from jax.experimental.pallas import tpu_sc as plsc

N = 10000
E = 320000
C = 128
NET = 7
NSEG = N * NET
NSEG_PAD = 70016
NC = 2
NS = 16
NW = NC * NS
EPT = E // NW
CH = 128
NFULL = EPT // CH
TAIL = EPT - NFULL * CH
ZPT = NSEG_PAD // NS
NPAD = 10240
ROWS_PT = NPAD // NS
EPS = 1e-5

_MESH = plsc.VectorSubcoreMesh(
    core_axis_name="c", subcore_axis_name="s", num_cores=NC, num_subcores=NS
)


def _counts_body(erow_hbm, etype_hbm, out_hbm,
                 rowb, typeb, segb, rowb_t, typeb_t, segb_t, ones, zbuf, hist):
    c = lax.axis_index("c")
    s = lax.axis_index("s")
    w = c * NS + s

    def _zinit(i, _):
        zbuf[pl.ds(i * 16, 16)] = jnp.zeros((16,), jnp.float32)
        return 0
    lax.fori_loop(0, (ZPT + 8) // 16, _zinit, 0)
    for i in range(CH // 16):
        ones[pl.ds(i * 16, 16)] = jnp.ones((16,), jnp.float32)
    pltpu.sync_copy(zbuf.at[pl.ds(0, ZPT)], hist.at[pl.ds(s * ZPT, ZPT)])
    plsc.subcore_barrier()

    def _chunk(g, _):
        base = w * EPT + g * CH
        pltpu.sync_copy(erow_hbm.at[pl.ds(base, CH)], rowb)
        pltpu.sync_copy(etype_hbm.at[pl.ds(base, CH)], typeb)
        for i in range(CH // 16):
            sl = pl.ds(i * 16, 16)
            segb[sl] = rowb[sl] * NET + typeb[sl]
        pltpu.sync_copy(ones, hist.at[segb], add=True)
        return 0
    lax.fori_loop(0, NFULL, _chunk, 0)

    base = w * EPT + NFULL * CH
    pltpu.sync_copy(erow_hbm.at[pl.ds(base, TAIL)], rowb_t)
    pltpu.sync_copy(etype_hbm.at[pl.ds(base, TAIL)], typeb_t)
    segb_t[...] = rowb_t[...] * NET + typeb_t[...]
    pltpu.sync_copy(ones.at[pl.ds(0, TAIL)], hist.at[segb_t], add=True)

    plsc.subcore_barrier()
    pltpu.sync_copy(hist.at[pl.ds(s * ZPT, ZPT)], zbuf.at[pl.ds(0, ZPT)])
    pltpu.sync_copy(zbuf.at[pl.ds(0, ZPT)],
                    out_hbm.at[pl.ds(c * NSEG_PAD + s * ZPT, ZPT)])


_SC_PARAMS = pltpu.CompilerParams(needs_layout_passes=False)

_counts = pl.kernel(
    _counts_body,
    compiler_params=_SC_PARAMS,
    out_type=jax.ShapeDtypeStruct((NC * NSEG_PAD,), jnp.float32),
    mesh=_MESH,
    scratch_types=[
        pltpu.VMEM((CH,), jnp.int32),
        pltpu.VMEM((CH,), jnp.int32),
        pltpu.VMEM((CH,), jnp.int32),
        pltpu.VMEM((TAIL,), jnp.int32),
        pltpu.VMEM((TAIL,), jnp.int32),
        pltpu.VMEM((TAIL,), jnp.int32),
        pltpu.VMEM((CH,), jnp.float32),
        pltpu.VMEM((ZPT + 8,), jnp.float32),
        pltpu.VMEM_SHARED((NSEG_PAD,), jnp.float32),
    ],
)


def _scatter_body(z_hbm, inv_hbm, erow_hbm, ecol_hbm, etype_hbm, out_hbm,
                  rowb, colb, typeb, zidx, segb, scaleb,
                  rowb_t, colb_t, typeb_t, zidx_t, segb_t, scaleb_t,
                  gbuf, sem, sem2, acc):
    c = lax.axis_index("c")
    s = lax.axis_index("s")
    w = c * NS + s

    def _zg(i, _):
        for j in range(C // 16):
            gbuf[i, pl.ds(j * 16, 16)] = jnp.zeros((16,), jnp.float32)
        return 0
    lax.fori_loop(0, CH, _zg, 0)
    for k in range(ROWS_PT // CH):
        pltpu.sync_copy(gbuf, acc.at[pl.ds(s * ROWS_PT + k * CH, CH)])
    plsc.subcore_barrier()

    def _chunk(g, _):
        base = w * EPT + g * CH
        pltpu.sync_copy(erow_hbm.at[pl.ds(base, CH)], rowb)
        pltpu.sync_copy(ecol_hbm.at[pl.ds(base, CH)], colb)
        pltpu.sync_copy(etype_hbm.at[pl.ds(base, CH)], typeb)
        for i in range(CH // 16):
            sl = pl.ds(i * 16, 16)
            t = typeb[sl]
            zidx[sl] = colb[sl] * NET + t
            segb[sl] = rowb[sl] * NET + t
        cp1 = pltpu.async_copy(z_hbm.at[zidx], gbuf, sem)
        cp2 = pltpu.async_copy(inv_hbm.at[segb], scaleb.at[pl.ds(0, CH)], sem2)
        cp1.wait()
        cp2.wait()

        def _scale(e, _):
            sc = scaleb[pl.ds(e, 16)][0]
            for j in range(C // 16):
                slj = pl.ds(j * 16, 16)
                gbuf[e, slj] = gbuf[e, slj] * sc
            return 0
        lax.fori_loop(0, CH, _scale, 0)
        pltpu.sync_copy(gbuf, acc.at[rowb], add=True)
        return 0
    lax.fori_loop(0, NFULL, _chunk, 0)

    base = w * EPT + NFULL * CH
    pltpu.sync_copy(erow_hbm.at[pl.ds(base, TAIL)], rowb_t)
    pltpu.sync_copy(ecol_hbm.at[pl.ds(base, TAIL)], colb_t)
    pltpu.sync_copy(etype_hbm.at[pl.ds(base, TAIL)], typeb_t)
    t = typeb_t[...]
    zidx_t[...] = colb_t[...] * NET + t
    segb_t[...] = rowb_t[...] * NET + t
    cp1 = pltpu.async_copy(z_hbm.at[zidx_t], gbuf.at[pl.ds(0, TAIL)], sem)
    cp2 = pltpu.async_copy(inv_hbm.at[segb_t], scaleb_t.at[pl.ds(0, TAIL)], sem2)
    cp1.wait()
    cp2.wait()

    def _scale_t(e, _):
        sc = scaleb_t[pl.ds(e, 16)][0]
        for j in range(C // 16):
            slj = pl.ds(j * 16, 16)
            gbuf[e, slj] = gbuf[e, slj] * sc
        return 0
    lax.fori_loop(0, TAIL, _scale_t, 0)
    pltpu.sync_copy(gbuf.at[pl.ds(0, TAIL)], acc.at[rowb_t], add=True)

    plsc.subcore_barrier()
    for k in range(ROWS_PT // CH):
        off = s * ROWS_PT + k * CH
        pltpu.sync_copy(acc.at[pl.ds(off, CH)], gbuf)
        pltpu.sync_copy(gbuf, out_hbm.at[c, pl.ds(off, CH), :])


_scatter = pl.kernel(
    _scatter_body,
    compiler_params=_SC_PARAMS,
    out_type=jax.ShapeDtypeStruct((NC, NPAD, C), jnp.float32),
    mesh=_MESH,
    scratch_types=[
        pltpu.VMEM((CH,), jnp.int32),
        pltpu.VMEM((CH,), jnp.int32),
        pltpu.VMEM((CH,), jnp.int32),
        pltpu.VMEM((CH,), jnp.int32),
        pltpu.VMEM((CH,), jnp.int32),
        pltpu.VMEM((CH + 16,), jnp.float32),
        pltpu.VMEM((TAIL,), jnp.int32),
        pltpu.VMEM((TAIL,), jnp.int32),
        pltpu.VMEM((TAIL,), jnp.int32),
        pltpu.VMEM((TAIL,), jnp.int32),
        pltpu.VMEM((TAIL,), jnp.int32),
        pltpu.VMEM((TAIL + 16,), jnp.float32),
        pltpu.VMEM((CH, C), jnp.float32),
        pltpu.SemaphoreType.DMA,
        pltpu.SemaphoreType.DMA,
        pltpu.VMEM_SHARED((NPAD, C), jnp.float32),
    ],
)


_BM = 1000


def _mm_body(x_ref, w_ref, z_ref, base_ref):
    xb = x_ref[...]
    zs = [jnp.dot(xb, w_ref[t * C:(t + 1) * C, :],
                  preferred_element_type=jnp.float32) for t in range(NET)]
    z_ref[...] = jnp.concatenate(zs, axis=1)
    base_ref[...] = zs[NET - 1]


_mm = pl.pallas_call(
    _mm_body,
    grid=(N // _BM,),
    in_specs=[pl.BlockSpec((_BM, C), lambda i: (i, 0)),
              pl.BlockSpec((NET * C, C), lambda i: (0, 0))],
    out_specs=[pl.BlockSpec((_BM, NET * C), lambda i: (i, 0)),
               pl.BlockSpec((_BM, C), lambda i: (i, 0))],
    out_shape=[jax.ShapeDtypeStruct((N, NET * C), jnp.float32),
               jax.ShapeDtypeStruct((N, C), jnp.float32)],
)


def _inv_body(cnt_ref, inv_ref):
    ctot = cnt_ref[0] + cnt_ref[1]
    rows = NSEG_PAD // 128
    rr = lax.broadcasted_iota(jnp.int32, (rows, 128), 0)
    cc = lax.broadcasted_iota(jnp.int32, (rows, 128), 1)
    sidx = rr * 128 + cc
    m6 = (sidx % NET) == (NET - 1)
    inv = 1.0 / jnp.maximum(ctot, 1.0)
    inv_ref[...] = jnp.where(m6, 0.0, inv)


_inv = pl.pallas_call(
    _inv_body,
    out_shape=jax.ShapeDtypeStruct((NSEG_PAD // 128, 128), jnp.float32),
)


def _bn_body(base_ref, accs_ref, g_ref, b_ref, y_ref):
    out = base_ref[...] + accs_ref[0] + accs_ref[1]
    mu = jnp.mean(out, axis=0, keepdims=True)
    d = out - mu
    var = jnp.mean(d * d, axis=0, keepdims=True)
    y_ref[...] = d * lax.rsqrt(var + EPS) * g_ref[...] + b_ref[...]


_bn = pl.pallas_call(
    _bn_body,
    grid=(1,),
    in_specs=[pl.BlockSpec((N, C), lambda i: (0, 0)),
              pl.BlockSpec((NC, N, C), lambda i: (0, 0, 0)),
              pl.BlockSpec((1, C), lambda i: (0, 0)),
              pl.BlockSpec((1, C), lambda i: (0, 0))],
    out_specs=pl.BlockSpec((N, C), lambda i: (0, 0)),
    out_shape=jax.ShapeDtypeStruct((N, C), jnp.float32),
)


def kernel(x, W, gamma, beta, edge_idx, edge_type):
    edge_idx = edge_idx.astype(jnp.int32)
    edge_type = edge_type.astype(jnp.int32)
    row, col = edge_idx[0], edge_idx[1]
    cnt = _counts(row, edge_type)
    Zf, base = _mm(x, W)
    inv = _inv(cnt.reshape(NC, NSEG_PAD // 128, 128)).reshape(NSEG_PAD)
    Z = Zf.reshape(N * NET, C)
    accs = _scatter(Z, inv, row, col, edge_type)
    return _bn(base, accs, gamma.reshape(1, C), beta.reshape(1, C))
